# Initial kernel scaffold; baseline (speedup 1.0000x reference)
import jax
import jax.numpy as jnp
from jax.experimental import pallas as pl


def kernel(ids, table, W, b):
    raise NotImplementedError("write your pallas kernel here")


# speedup vs baseline: 19.0443x; 19.0443x over previous
import functools

import jax
import jax.numpy as jnp
from jax import lax
from jax.experimental import pallas as pl
from jax.experimental.pallas import tpu as pltpu
from jax.experimental.pallas import tpu_sc as plsc

_VOCAB = 1000000
_EMB = 32
_B = 16384
_L = 200
_PAD = 0

_ROWS_A = 5000
_GRID_A = _VOCAB // _ROWS_A

_NC, _NS = 2, 16
_NW = _NC * _NS
_COLS_W = _B // _NW
_CH = 128
_NCH = _COLS_W // _CH


def _score_body(tab_ref, w_ref, out_ref):
    x = tab_ref[...]
    w = w_ref[...].reshape(1, 1, _EMB)
    s = jnp.sum(x * w, axis=-1)
    i = pl.program_id(0)
    col = lax.broadcasted_iota(jnp.int32, (1, _ROWS_A), 1)
    s = jnp.where((i == 0) & (col == _PAD), 0.0, s)
    out_ref[...] = (s * (1.0 / _L))[:, None, :]


def _scores(table, W):
    tab3 = table.reshape(_GRID_A, _ROWS_A, _EMB)
    out3 = pl.pallas_call(
        _score_body,
        grid=(_GRID_A,),
        in_specs=[
            pl.BlockSpec((1, _ROWS_A, _EMB), lambda i: (i, 0, 0)),
            pl.BlockSpec((1, _EMB), lambda i: (0, 0)),
        ],
        out_specs=pl.BlockSpec((1, 1, _ROWS_A), lambda i: (i, 0, 0)),
        out_shape=jax.ShapeDtypeStruct((_GRID_A, 1, _ROWS_A), jnp.float32),
    )(tab3, W)
    return out3.reshape(_VOCAB)


_CHW = _L * _CH


def _sc_pool_body(scores_hbm, ids_hbm, bvec_hbm, out_hbm,
                  idx_v, vals_v, out_v, b_v, sem):
    wid = lax.axis_index("s") * _NC + lax.axis_index("c")
    base = wid * _COLS_W
    pltpu.sync_copy(bvec_hbm, b_v)
    bv = b_v[...]

    def chunk(ci, carry):
        off = (wid * _NCH + ci) * _CHW
        pltpu.sync_copy(ids_hbm.at[pl.ds(off, _CHW)], idx_v)
        pltpu.async_copy(scores_hbm.at[idx_v], vals_v, sem).wait()

        for k in range(_CH // 16):
            def red(l, acc):
                return acc + vals_v[pl.ds(l * _CH + k * 16, 16)]
            acc = lax.fori_loop(0, _L, red, jnp.zeros((16,), jnp.float32))
            z = acc + bv
            y = 1.0 / (1.0 + jnp.exp(-z))
            out_v[pl.ds(ci * _CH + k * 16, 16)] = y
        return carry

    lax.fori_loop(0, _NCH, chunk, 0)
    pltpu.sync_copy(out_v, out_hbm.at[pl.ds(base, _COLS_W)])


def _sc_pool(scores, ids_t, bvec):
    mesh = plsc.VectorSubcoreMesh(core_axis_name="c", subcore_axis_name="s")
    f = pl.kernel(
        _sc_pool_body,
        out_type=jax.ShapeDtypeStruct((_B,), jnp.float32),
        mesh=mesh,
        scratch_types=[
            pltpu.VMEM((_CHW,), jnp.int32),
            pltpu.VMEM((_CHW,), jnp.float32),
            pltpu.VMEM((_COLS_W,), jnp.float32),
            pltpu.VMEM((16,), jnp.float32),
            pltpu.SemaphoreType.DMA,
        ],
    )
    return f(scores, ids_t, bvec)


def kernel(ids, table, W, b):
    scores = _scores(table.astype(jnp.float32), W.astype(jnp.float32))
    n_chunks = _B // _CH
    ids_r = (ids.astype(jnp.int32).T
             .reshape(_L, n_chunks, _CH)
             .transpose(1, 0, 2)
             .reshape(-1))
    bvec = jnp.broadcast_to(b.astype(jnp.float32), (16,))
    out_flat = _sc_pool(scores, ids_r, bvec)
    return out_flat.reshape(_B, 1)


# speedup vs baseline: 19.5181x; 1.0249x over previous
import functools

import jax
import jax.numpy as jnp
from jax import lax
from jax.experimental import pallas as pl
from jax.experimental.pallas import tpu as pltpu
from jax.experimental.pallas import tpu_sc as plsc

_VOCAB = 1000000
_EMB = 32
_B = 16384
_L = 200
_PAD = 0

_ROWS_A = 5000
_GRID_A = _VOCAB // _ROWS_A

_NC, _NS = 2, 16
_NW = _NC * _NS
_COLS_W = _B // _NW
_CH = 128
_NCH = _COLS_W // _CH


def _score_body(tab_ref, w_ref, out_ref):
    x = tab_ref[...]
    w = w_ref[...].reshape(1, 1, _EMB)
    s = jnp.sum(x * w, axis=-1)
    i = pl.program_id(0)
    col = lax.broadcasted_iota(jnp.int32, (1, _ROWS_A), 1)
    s = jnp.where((i == 0) & (col == _PAD), 0.0, s)
    out_ref[...] = (s * (1.0 / _L))[:, None, :]


def _scores(table, W):
    tab3 = table.reshape(_GRID_A, _ROWS_A, _EMB)
    out3 = pl.pallas_call(
        _score_body,
        grid=(_GRID_A,),
        in_specs=[
            pl.BlockSpec((1, _ROWS_A, _EMB), lambda i: (i, 0, 0)),
            pl.BlockSpec((1, _EMB), lambda i: (0, 0)),
        ],
        out_specs=pl.BlockSpec((1, 1, _ROWS_A), lambda i: (i, 0, 0)),
        out_shape=jax.ShapeDtypeStruct((_GRID_A, 1, _ROWS_A), jnp.float32),
    )(tab3, W)
    return out3.reshape(_VOCAB)


_CHW = _L * _CH
_NCHUNKS = _B // _CH


def _tr_body(ids_ref, out_ref):
    x = ids_ref[...]
    out_ref[...] = x.T[None]


def _permute_ids(ids):
    return pl.pallas_call(
        _tr_body,
        grid=(_NCHUNKS,),
        in_specs=[pl.BlockSpec((_CH, _L), lambda c: (c, 0))],
        out_specs=pl.BlockSpec((1, _L, _CH), lambda c: (c, 0, 0)),
        out_shape=jax.ShapeDtypeStruct((_NCHUNKS, _L, _CH), jnp.int32),
    )(ids)


def _sc_pool_body(scores_hbm, ids_hbm, bvec_hbm, out_hbm,
                  idx_v, vals_v, out_v, b_v, sem):
    wid = lax.axis_index("s") * _NC + lax.axis_index("c")
    base = wid * _COLS_W
    pltpu.sync_copy(bvec_hbm, b_v)
    bv = b_v[...]

    def chunk(ci, carry):
        off = (wid * _NCH + ci) * _CHW
        pltpu.sync_copy(ids_hbm.at[pl.ds(off, _CHW)], idx_v)
        pltpu.async_copy(scores_hbm.at[idx_v], vals_v, sem).wait()

        for k in range(_CH // 16):
            def red(l, acc):
                return acc + vals_v[pl.ds(l * _CH + k * 16, 16)]
            acc = lax.fori_loop(0, _L, red, jnp.zeros((16,), jnp.float32))
            z = acc + bv
            y = 1.0 / (1.0 + jnp.exp(-z))
            out_v[pl.ds(ci * _CH + k * 16, 16)] = y
        return carry

    lax.fori_loop(0, _NCH, chunk, 0)
    pltpu.sync_copy(out_v, out_hbm.at[pl.ds(base, _COLS_W)])


def _sc_pool(scores, ids_t, bvec):
    mesh = plsc.VectorSubcoreMesh(core_axis_name="c", subcore_axis_name="s")
    f = pl.kernel(
        _sc_pool_body,
        out_type=jax.ShapeDtypeStruct((_B,), jnp.float32),
        mesh=mesh,
        scratch_types=[
            pltpu.VMEM((_CHW,), jnp.int32),
            pltpu.VMEM((_CHW,), jnp.float32),
            pltpu.VMEM((_COLS_W,), jnp.float32),
            pltpu.VMEM((16,), jnp.float32),
            pltpu.SemaphoreType.DMA,
        ],
    )
    return f(scores, ids_t, bvec)


def kernel(ids, table, W, b):
    scores = _scores(table.astype(jnp.float32), W.astype(jnp.float32))
    ids_r = _permute_ids(ids.astype(jnp.int32)).reshape(-1)
    bvec = jnp.broadcast_to(b.astype(jnp.float32), (16,))
    out_flat = _sc_pool(scores, ids_r, bvec)
    return out_flat.reshape(_B, 1)
